# Initial kernel scaffold; baseline (speedup 1.0000x reference)
import jax
import jax.numpy as jnp
from jax.experimental import pallas as pl


def kernel(x, emb, W1, b1, g1, be1, W2, b2, g2, be2, W3, b3):
    raise NotImplementedError("write your pallas kernel here")


# speedup vs baseline: 7.6482x; 7.6482x over previous
import functools

import jax
import jax.numpy as jnp
from jax import lax
from jax.experimental import pallas as pl
from jax.experimental.pallas import tpu as pltpu
from jax.experimental.pallas import tpu_sc as plsc

B = 16384
NUM = 13
NCAT = 26
V = 100000
D = 16
H1, H2 = 512, 256
EPS = 1e-5

NC, NS = 2, 16
NW = NC * NS
TOTAL_ROWS = B * NCAT
ROWS_W = TOTAL_ROWS // NW
CH = 128
NCHUNK = ROWS_W // CH
RING = 8
PREF = 4
GROUPS = NCHUNK // RING


def _sc_gather(table, idx):
  mesh = plsc.VectorSubcoreMesh(core_axis_name="c", subcore_axis_name="s")

  @functools.partial(
      pl.kernel,
      out_type=jax.ShapeDtypeStruct((TOTAL_ROWS, D), jnp.float32),
      mesh=mesh,
      scratch_types=(
          [pltpu.VMEM((NCHUNK, CH), jnp.int32)]
          + [pltpu.VMEM((CH, D), jnp.float32) for _ in range(RING)]
          + [pltpu.SemaphoreType.DMA for _ in range(2 * RING)]
      ),
      compiler_params=pltpu.CompilerParams(use_tc_tiling_on_sc=False),
  )
  def gather_kernel(table_hbm, idx_hbm, out_hbm, idx_v, *bufs_and_sems):
    bufs = bufs_and_sems[:RING]
    gsem = bufs_and_sems[RING:2 * RING]
    wsem = bufs_and_sems[2 * RING:]
    wid = lax.axis_index("s") * NC + lax.axis_index("c")
    base = wid * ROWS_W

    pltpu.sync_copy(idx_hbm.at[wid], idx_v)

    def start_gather(j, s):
      pltpu.async_copy(table_hbm.at[idx_v.at[j]], bufs[s], gsem[s])

    def wait_gather(j, s):
      pltpu.make_async_copy(table_hbm.at[idx_v.at[j]], bufs[s], gsem[s]).wait()

    def start_write(j, s):
      pltpu.async_copy(bufs[s], out_hbm.at[pl.ds(base + j * CH, CH)], wsem[s])

    def wait_write(s):
      pltpu.make_async_copy(
          bufs[s], out_hbm.at[pl.ds(base, CH)], wsem[s]).wait()

    for b in range(PREF):
      start_gather(b, b)

    def group(j0, _):
      for b in range(RING):
        j = j0 * RING + b
        wait_gather(j, b)
        start_write(j, b)
        jn = j + PREF
        sn = (b + PREF) % RING

        @pl.when(jn < NCHUNK)
        def _():
          @pl.when(jn >= RING)
          def _():
            wait_write(sn)
          start_gather(jn, sn)
      return 0

    lax.fori_loop(0, GROUPS, group, 0, unroll=False)

    for b in range(RING):
      wait_write(b)

  return gather_kernel(table, idx)


def _mlp1(x_num, e, w1a, w1b, b1, bm):
  nb = B // bm

  def body(xn_ref, e_ref, w1a_ref, w1b_ref, b1_ref, h_ref, s_ref, ss_ref):
    i = pl.program_id(0)
    h = (
        jnp.dot(xn_ref[...], w1a_ref[...], preferred_element_type=jnp.float32)
        + jnp.dot(e_ref[...], w1b_ref[...], preferred_element_type=jnp.float32)
        + b1_ref[...]
    )
    h_ref[...] = h

    @pl.when(i == 0)
    def _():
      s_ref[...] = jnp.zeros_like(s_ref)
      ss_ref[...] = jnp.zeros_like(ss_ref)

    s_ref[...] += jnp.sum(h, axis=0, keepdims=True)
    ss_ref[...] += jnp.sum(h * h, axis=0, keepdims=True)

  return pl.pallas_call(
      body,
      grid=(nb,),
      in_specs=[
          pl.BlockSpec((bm, NUM), lambda i: (i, 0)),
          pl.BlockSpec((bm, NCAT * D), lambda i: (i, 0)),
          pl.BlockSpec((NUM, H1), lambda i: (0, 0)),
          pl.BlockSpec((NCAT * D, H1), lambda i: (0, 0)),
          pl.BlockSpec((1, H1), lambda i: (0, 0)),
      ],
      out_specs=[
          pl.BlockSpec((bm, H1), lambda i: (i, 0)),
          pl.BlockSpec((1, H1), lambda i: (0, 0)),
          pl.BlockSpec((1, H1), lambda i: (0, 0)),
      ],
      out_shape=[
          jax.ShapeDtypeStruct((B, H1), jnp.float32),
          jax.ShapeDtypeStruct((1, H1), jnp.float32),
          jax.ShapeDtypeStruct((1, H1), jnp.float32),
      ],
  )(x_num, e, w1a, w1b, b1)


def _mlp2(h1, s1, ss1, g1, be1, w2, b2, bm):
  nb = B // bm

  def body(h1_ref, s_ref, ss_ref, g_ref, be_ref, w2_ref, b2_ref,
           h_ref, s2_ref, ss2_ref):
    i = pl.program_id(0)
    mean = s_ref[...] * (1.0 / B)
    var = ss_ref[...] * (1.0 / B) - mean * mean
    inv = lax.rsqrt(var + EPS)
    scale = g_ref[...] * inv
    shift = be_ref[...] - mean * scale
    a = jnp.maximum(h1_ref[...] * scale + shift, 0.0)
    h = jnp.dot(a, w2_ref[...], preferred_element_type=jnp.float32) + b2_ref[...]
    h_ref[...] = h

    @pl.when(i == 0)
    def _():
      s2_ref[...] = jnp.zeros_like(s2_ref)
      ss2_ref[...] = jnp.zeros_like(ss2_ref)

    s2_ref[...] += jnp.sum(h, axis=0, keepdims=True)
    ss2_ref[...] += jnp.sum(h * h, axis=0, keepdims=True)

  return pl.pallas_call(
      body,
      grid=(nb,),
      in_specs=[
          pl.BlockSpec((bm, H1), lambda i: (i, 0)),
          pl.BlockSpec((1, H1), lambda i: (0, 0)),
          pl.BlockSpec((1, H1), lambda i: (0, 0)),
          pl.BlockSpec((1, H1), lambda i: (0, 0)),
          pl.BlockSpec((1, H1), lambda i: (0, 0)),
          pl.BlockSpec((H1, H2), lambda i: (0, 0)),
          pl.BlockSpec((1, H2), lambda i: (0, 0)),
      ],
      out_specs=[
          pl.BlockSpec((bm, H2), lambda i: (i, 0)),
          pl.BlockSpec((1, H2), lambda i: (0, 0)),
          pl.BlockSpec((1, H2), lambda i: (0, 0)),
      ],
      out_shape=[
          jax.ShapeDtypeStruct((B, H2), jnp.float32),
          jax.ShapeDtypeStruct((1, H2), jnp.float32),
          jax.ShapeDtypeStruct((1, H2), jnp.float32),
      ],
  )(h1, s1, ss1, g1, be1, w2, b2)


def _mlp3(h2, s2, ss2, g2, be2, w3row, b3, bm):
  nb = B // bm

  def body(h2_ref, s_ref, ss_ref, g_ref, be_ref, w3_ref, b3_ref, out_ref):
    mean = s_ref[...] * (1.0 / B)
    var = ss_ref[...] * (1.0 / B) - mean * mean
    inv = lax.rsqrt(var + EPS)
    scale = g_ref[...] * inv
    shift = be_ref[...] - mean * scale
    a = jnp.maximum(h2_ref[...] * scale + shift, 0.0)
    out_ref[...] = (
        jnp.sum(a * w3_ref[...], axis=1, keepdims=True) + b3_ref[...]
    )

  return pl.pallas_call(
      body,
      grid=(nb,),
      in_specs=[
          pl.BlockSpec((bm, H2), lambda i: (i, 0)),
          pl.BlockSpec((1, H2), lambda i: (0, 0)),
          pl.BlockSpec((1, H2), lambda i: (0, 0)),
          pl.BlockSpec((1, H2), lambda i: (0, 0)),
          pl.BlockSpec((1, H2), lambda i: (0, 0)),
          pl.BlockSpec((1, H2), lambda i: (0, 0)),
          pl.BlockSpec((1, 1), lambda i: (0, 0)),
      ],
      out_specs=pl.BlockSpec((bm, 1), lambda i: (i, 0)),
      out_shape=jax.ShapeDtypeStruct((B, 1), jnp.float32),
  )(h2, s2, ss2, g2, be2, w3row, b3)


def kernel(x, emb, W1, b1, g1, be1, W2, b2, g2, be2, W3, b3):
  x_num = x[:, :NUM]
  cat = x[:, NUM:].astype(jnp.int32)
  flat_idx = (cat + jnp.arange(NCAT, dtype=jnp.int32)[None, :] * V).reshape(
      NW, NCHUNK, CH)
  table = emb.reshape(NCAT * V, D)

  e = _sc_gather(table, flat_idx).reshape(B, NCAT * D)

  w1a = W1[:NUM]
  w1b = W1[NUM:]
  h1, s1, ss1 = _mlp1(x_num, e, w1a, w1b, b1.reshape(1, H1), bm=2048)
  h2, s2, ss2 = _mlp2(h1, s1, ss1, g1.reshape(1, H1), be1.reshape(1, H1),
                      W2, b2.reshape(1, H2), bm=2048)
  out = _mlp3(h2, s2, ss2, g2.reshape(1, H2), be2.reshape(1, H2),
              W3.reshape(1, H2), b3.reshape(1, 1), bm=2048)
  return out
